# baseline (device time: 23019 ns/iter reference)
import jax
import jax.numpy as jnp
from jax import lax
from jax.experimental import pallas as pl
from jax.experimental.pallas import tpu as pltpu

N_DEV = 4


def kernel(x, Wq, K_ext, V_ext, Wo):
    B_loc, Sq, E = x.shape
    _, wq_cols = Wq.shape
    Bg, Skv, Hq, Dh = K_ext.shape
    H_loc = wq_cols // Dh
    Eo = Wo.shape[1]
    BS = B_loc * Sq
    HH = wq_cols // 2
    CR = 2 * wq_cols
    CH = CR // 2

    my_pos = lax.axis_index("i")

    xf = x.reshape(BS, E).astype(jnp.bfloat16)
    WqT = jnp.transpose(Wq)
    Wcomb = jnp.concatenate(
        [WqT[0:HH], Wo[0:HH], WqT[HH:2 * HH], Wo[HH:2 * HH]], axis=0,
    ).astype(jnp.bfloat16)
    Kb = lax.dynamic_slice_in_dim(K_ext, B_loc * my_pos, B_loc, axis=0)
    Vb = lax.dynamic_slice_in_dim(V_ext, B_loc * my_pos, B_loc, axis=0)
    Kb = jnp.transpose(Kb, (2, 0, 1, 3)).reshape(Hq * BS, Dh)
    Vb = jnp.transpose(Vb, (2, 0, 1, 3)).reshape(Hq * BS, Dh)
    Kb = Kb.astype(jnp.bfloat16)
    Vb = Vb.astype(jnp.bfloat16)

    def body(x_ref, wc_ref, k_ref, v_ref, out_ref,
             cg, ctx_ref, send_sems, recv_sems):
        my = lax.axis_index("i")
        left = lax.rem(my + N_DEV - 1, N_DEV)
        right = lax.rem(my + 1, N_DEV)

        cg[0, :, :] = wc_ref[:, :]

        barrier = pltpu.get_barrier_semaphore()
        for nbr in (left, right):
            pl.semaphore_signal(
                barrier, inc=1,
                device_id=(nbr,), device_id_type=pl.DeviceIdType.MESH,
            )
        pl.semaphore_wait(barrier, 2)

        def half(slot, t):
            return cg.at[slot, pl.ds(t * CH, CH)]

        def copy(src, dst, sem_idx, target):
            return pltpu.make_async_remote_copy(
                src_ref=src, dst_ref=dst,
                send_sem=send_sems.at[sem_idx], recv_sem=recv_sems.at[sem_idx],
                device_id=(target,), device_id_type=pl.DeviceIdType.MESH,
            )

        a_sends = [
            copy(half(0, 0), half(1, 0), 0, right),
            copy(half(0, 1), half(2, 1), 1, left),
            copy(half(0, 1), half(1, 1), 2, right),
            copy(half(0, 0), half(2, 0), 3, left),
        ]
        for r in a_sends:
            r.start()

        def compute_half(slot, j, t):
            q_half = lax.dot_general(
                x_ref[:, :], cg[slot, t * CH:t * CH + HH, :],
                (((1,), (1,)), ((), ())),
                preferred_element_type=jnp.float32)
            q16 = q_half.astype(jnp.bfloat16)
            for b in range(B_loc):
                for u in range(2):
                    hh = 2 * t + u
                    q = q16[b * Sq:(b + 1) * Sq, u * Dh:(u + 1) * Dh]
                    off = ((j * H_loc + hh) * B_loc + b) * Sq
                    k = k_ref[pl.ds(off, Skv), :]
                    v = v_ref[pl.ds(off, Skv), :]
                    s = lax.dot_general(
                        q, k, (((1,), (1,)), ((), ())),
                        preferred_element_type=jnp.float32) * 0.125
                    w = jnp.exp(s)
                    w = (w / jnp.sum(w, axis=1, keepdims=True)).astype(
                        jnp.bfloat16)
                    ctx_ref[b * Sq:(b + 1) * Sq,
                            t * HH + u * Dh:t * HH + (u + 1) * Dh] = (
                        jnp.dot(w, v, preferred_element_type=jnp.float32)
                        .astype(jnp.bfloat16))
            part = jnp.dot(ctx_ref[:, t * HH:(t + 1) * HH],
                           cg[slot, t * CH + HH:(t + 1) * CH, :],
                           preferred_element_type=jnp.float32)
            if slot == 0 and t == 0:
                out_ref[:, :] = part
            else:
                out_ref[:, :] += part

        compute_half(0, my, 0)
        compute_half(0, my, 1)

        a_sends[0].wait_recv()
        fwd_r = copy(half(1, 0), half(3, 0), 4, right)
        fwd_r.start()
        compute_half(1, left, 0)

        a_sends[1].wait_recv()
        fwd_l = copy(half(2, 1), half(3, 1), 5, left)
        fwd_l.start()
        compute_half(2, right, 1)

        a_sends[2].wait_recv()
        compute_half(1, left, 1)
        a_sends[3].wait_recv()
        compute_half(2, right, 0)

        opp = lax.rem(my + 2, N_DEV)
        fwd_r.wait_recv()
        compute_half(3, opp, 0)
        fwd_l.wait_recv()
        compute_half(3, opp, 1)

        for r in a_sends + [fwd_r, fwd_l]:
            r.wait_send()

    out_flat = pl.pallas_call(
        body,
        out_shape=jax.ShapeDtypeStruct((BS, Eo), jnp.float32),
        in_specs=[pl.BlockSpec(memory_space=pltpu.VMEM)] * 4,
        out_specs=pl.BlockSpec(memory_space=pltpu.VMEM),
        scratch_shapes=[
            pltpu.VMEM((N_DEV, CR, E), jnp.bfloat16),
            pltpu.VMEM((BS, wq_cols), jnp.bfloat16),
            pltpu.SemaphoreType.DMA((6,)),
            pltpu.SemaphoreType.DMA((6,)),
        ],
        compiler_params=pltpu.CompilerParams(collective_id=0),
    )(xf, Wcomb, Kb, Vb)

    return out_flat.reshape(B_loc, Sq, Eo)


# device time: 20969 ns/iter; 1.0978x vs baseline; 1.0978x over previous
import jax
import jax.numpy as jnp
from jax import lax
from jax.experimental import pallas as pl
from jax.experimental.pallas import tpu as pltpu

N_DEV = 4


def kernel(x, Wq, K_ext, V_ext, Wo):
    B_loc, Sq, E = x.shape
    _, wq_cols = Wq.shape
    Bg, Skv, Hq, Dh = K_ext.shape
    H_loc = wq_cols // Dh
    Eo = Wo.shape[1]
    BS = B_loc * Sq
    HH = wq_cols // 2
    CR = 2 * wq_cols
    CH = CR // 2

    my_pos = lax.axis_index("i")

    xf = x.reshape(BS, E)
    WqT = jnp.transpose(Wq)
    Kb = lax.dynamic_slice_in_dim(K_ext, B_loc * my_pos, B_loc, axis=0)
    Vb = lax.dynamic_slice_in_dim(V_ext, B_loc * my_pos, B_loc, axis=0)
    Kb = jnp.transpose(Kb, (2, 0, 1, 3)).reshape(Hq * BS, Dh)
    Vb = jnp.transpose(Vb, (2, 0, 1, 3)).reshape(Hq * BS, Dh)
    Kb = Kb.astype(jnp.bfloat16)
    Vb = Vb.astype(jnp.bfloat16)

    def body(x_ref, wqt_ref, k_ref, v_ref, wo_ref, out_ref,
             cg, x16, ctx_ref, send_sems, recv_sems):
        my = lax.axis_index("i")
        left = lax.rem(my + N_DEV - 1, N_DEV)
        right = lax.rem(my + 1, N_DEV)

        x16[:, :] = x_ref[:, :].astype(jnp.bfloat16)
        for t in range(2):
            cg[0, t * CH:t * CH + HH, :] = (
                wqt_ref[t * HH:(t + 1) * HH, :].astype(jnp.bfloat16))
            cg[0, t * CH + HH:(t + 1) * CH, :] = (
                wo_ref[t * HH:(t + 1) * HH, :].astype(jnp.bfloat16))

        barrier = pltpu.get_barrier_semaphore()
        for nbr in (left, right):
            pl.semaphore_signal(
                barrier, inc=1,
                device_id=(nbr,), device_id_type=pl.DeviceIdType.MESH,
            )
        pl.semaphore_wait(barrier, 2)

        def half(slot, t):
            return cg.at[slot, pl.ds(t * CH, CH)]

        def copy(src, dst, sem_idx, target):
            return pltpu.make_async_remote_copy(
                src_ref=src, dst_ref=dst,
                send_sem=send_sems.at[sem_idx], recv_sem=recv_sems.at[sem_idx],
                device_id=(target,), device_id_type=pl.DeviceIdType.MESH,
            )

        a_sends = [
            copy(half(0, 0), half(1, 0), 0, right),
            copy(half(0, 1), half(2, 1), 1, left),
            copy(half(0, 1), half(1, 1), 2, right),
            copy(half(0, 0), half(2, 0), 3, left),
        ]
        for r in a_sends:
            r.start()

        def compute_half(slot, j, t):
            q_half = lax.dot_general(
                x16[:, :], cg[slot, t * CH:t * CH + HH, :],
                (((1,), (1,)), ((), ())),
                preferred_element_type=jnp.float32)
            q16 = q_half.astype(jnp.bfloat16)
            for b in range(B_loc):
                for u in range(2):
                    hh = 2 * t + u
                    q = q16[b * Sq:(b + 1) * Sq, u * Dh:(u + 1) * Dh]
                    off = ((j * H_loc + hh) * B_loc + b) * Sq
                    k = k_ref[pl.ds(off, Skv), :]
                    v = v_ref[pl.ds(off, Skv), :]
                    s = lax.dot_general(
                        q, k, (((1,), (1,)), ((), ())),
                        preferred_element_type=jnp.float32) * 0.125
                    w = jnp.exp(s)
                    w = (w / jnp.sum(w, axis=1, keepdims=True)).astype(
                        jnp.bfloat16)
                    ctx_ref[b * Sq:(b + 1) * Sq,
                            t * HH + u * Dh:t * HH + (u + 1) * Dh] = (
                        jnp.dot(w, v, preferred_element_type=jnp.float32)
                        .astype(jnp.bfloat16))
            part = jnp.dot(ctx_ref[:, t * HH:(t + 1) * HH],
                           cg[slot, t * CH + HH:(t + 1) * CH, :],
                           preferred_element_type=jnp.float32)
            if slot == 0 and t == 0:
                out_ref[:, :] = part
            else:
                out_ref[:, :] += part

        compute_half(0, my, 0)
        compute_half(0, my, 1)

        a_sends[0].wait_recv()
        fwd_r = copy(half(1, 0), half(3, 0), 4, right)
        fwd_r.start()
        compute_half(1, left, 0)

        a_sends[1].wait_recv()
        fwd_l = copy(half(2, 1), half(3, 1), 5, left)
        fwd_l.start()
        compute_half(2, right, 1)

        a_sends[2].wait_recv()
        compute_half(1, left, 1)
        a_sends[3].wait_recv()
        compute_half(2, right, 0)

        opp = lax.rem(my + 2, N_DEV)
        fwd_r.wait_recv()
        compute_half(3, opp, 0)
        fwd_l.wait_recv()
        compute_half(3, opp, 1)

        for r in a_sends + [fwd_r, fwd_l]:
            r.wait_send()

    out_flat = pl.pallas_call(
        body,
        out_shape=jax.ShapeDtypeStruct((BS, Eo), jnp.float32),
        in_specs=[pl.BlockSpec(memory_space=pltpu.VMEM)] * 5,
        out_specs=pl.BlockSpec(memory_space=pltpu.VMEM),
        scratch_shapes=[
            pltpu.VMEM((N_DEV, CR, E), jnp.bfloat16),
            pltpu.VMEM((BS, E), jnp.bfloat16),
            pltpu.VMEM((BS, wq_cols), jnp.bfloat16),
            pltpu.SemaphoreType.DMA((6,)),
            pltpu.SemaphoreType.DMA((6,)),
        ],
        compiler_params=pltpu.CompilerParams(collective_id=0),
    )(xf, WqT, Kb, Vb, Wo)

    return out_flat.reshape(B_loc, Sq, Eo)
